# Initial kernel scaffold; baseline (speedup 1.0000x reference)
import jax
import jax.numpy as jnp
from jax.experimental import pallas as pl


def kernel(x, gate_w, gate_b, w1, b1, w2, b2, w3, b3, sw1, sb1, sw2, sb2, sw3, sb3):
    raise NotImplementedError("write your pallas kernel here")


# speedup vs baseline: 2.5332x; 2.5332x over previous
import functools
import jax
import jax.numpy as jnp
import numpy as np
from jax.experimental import pallas as pl
from jax.experimental.pallas import tpu as pltpu

DIM = 1024
INTER = 512
E = 8
NG = 4
TOPK = 2
T = 2048
SHARED_INTER = 2 * INTER

_NEG = -1e30


def _gate_kernel(x_ref, gw_ref, gb_ref, pm_ref, coef_ref):
    xf = x_ref[...]
    scores = jax.lax.dot_general(
        xf.astype(jnp.bfloat16), gw_ref[...].astype(jnp.bfloat16),
        (((1,), (1,)), ((), ())), preferred_element_type=jnp.float32)
    s = jax.nn.sigmoid(scores)
    sb = s + gb_ref[...]
    lane = jax.lax.broadcasted_iota(jnp.int32, (T, E), 1)
    gid = lane // 2
    glane = sb + jax.lax.dot_general(
        sb, pm_ref[...], (((1,), (0,)), ((), ())),
        preferred_element_type=jnp.float32,
        precision=jax.lax.Precision.HIGHEST)
    m1 = jnp.max(glane, axis=1, keepdims=True)
    g1 = jnp.min(jnp.where(glane >= m1, gid, 99), axis=1, keepdims=True)
    gl2 = jnp.where(gid == g1, _NEG, glane)
    m2 = jnp.max(gl2, axis=1, keepdims=True)
    g2 = jnp.min(jnp.where(gl2 >= m2, gid, 99), axis=1, keepdims=True)
    keep = (gid == g1) | (gid == g2)
    sk = jnp.where(keep, sb, _NEG)
    v1 = jnp.max(sk, axis=1, keepdims=True)
    e1 = jnp.min(jnp.where(sk >= v1, lane, 99), axis=1, keepdims=True)
    sk2 = jnp.where(lane == e1, _NEG, sk)
    v2 = jnp.max(sk2, axis=1, keepdims=True)
    e2 = jnp.min(jnp.where(sk2 >= v2, lane, 99), axis=1, keepdims=True)
    w1 = jnp.sum(jnp.where(lane == e1, s, 0.0), axis=1, keepdims=True)
    w2 = jnp.sum(jnp.where(lane == e2, s, 0.0), axis=1, keepdims=True)
    norm = w1 + w2
    coef_ref[...] = (jnp.where(lane == e1, w1, 0.0) +
                     jnp.where(lane == e2, w2, 0.0)) / norm


_PAIR_M = np.zeros((E, E), np.float32)
for _e in range(E):
    _PAIR_M[_e ^ 1, _e] = 1.0
_PAIR_M = jnp.asarray(_PAIR_M)


def _expert_kernel(coef_ref, x_ref, z_ref, w1_ref, w3_ref, w2_ref, out_ref):
    e = pl.program_id(0)
    xf = x_ref[...].astype(jnp.bfloat16)
    h1 = jax.lax.dot_general(xf, w1_ref[0].astype(jnp.bfloat16),
                             (((1,), (1,)), ((), ())),
                             preferred_element_type=jnp.float32)
    h3 = jax.lax.dot_general(xf, w3_ref[0].astype(jnp.bfloat16),
                             (((1,), (1,)), ((), ())),
                             preferred_element_type=jnp.float32)
    h = (h1 * jax.nn.sigmoid(h1)) * h3
    out = jax.lax.dot_general(h.astype(jnp.bfloat16),
                              w2_ref[0].astype(jnp.bfloat16),
                              (((1,), (1,)), ((), ())),
                              preferred_element_type=jnp.float32)
    lane = jax.lax.broadcasted_iota(jnp.int32, (T, E), 1)
    c = jnp.sum(jnp.where(lane == e, coef_ref[...], 0.0), axis=1,
                keepdims=True)

    @pl.when(e == 0)
    def _():
        out_ref[...] = z_ref[...] + c * out

    @pl.when(e != 0)
    def _():
        out_ref[...] = out_ref[...] + c * out


def _shared_kernel(x_ref, sw1_ref, sw3_ref, sw2_ref, z_ref):
    xf = x_ref[...].astype(jnp.bfloat16)
    h1 = jax.lax.dot_general(xf, sw1_ref[...].astype(jnp.bfloat16),
                             (((1,), (1,)), ((), ())),
                             preferred_element_type=jnp.float32)
    h3 = jax.lax.dot_general(xf, sw3_ref[...].astype(jnp.bfloat16),
                             (((1,), (1,)), ((), ())),
                             preferred_element_type=jnp.float32)
    h = (h1 * jax.nn.sigmoid(h1)) * h3
    z_ref[...] = jax.lax.dot_general(h.astype(jnp.bfloat16),
                                     sw2_ref[...].astype(jnp.bfloat16),
                                     (((1,), (1,)), ((), ())),
                                     preferred_element_type=jnp.float32)


def kernel(x, gate_w, gate_b, w1, b1, w2, b2, w3, b3,
           sw1, sb1, sw2, sb2, sw3, sb3):
    shape = x.shape
    xf = x.reshape(T, DIM)

    coef = pl.pallas_call(
        _gate_kernel,
        out_shape=jax.ShapeDtypeStruct((T, E), jnp.float32),
    )(xf, gate_w, gate_b.reshape(1, E), _PAIR_M)

    z = pl.pallas_call(
        _shared_kernel,
        out_shape=jax.ShapeDtypeStruct((T, DIM), jnp.float32),
    )(xf, sw1, sw3, sw2)

    y = pl.pallas_call(
        _expert_kernel,
        grid=(E,),
        in_specs=[
            pl.BlockSpec((T, E), lambda e: (0, 0)),
            pl.BlockSpec((T, DIM), lambda e: (0, 0)),
            pl.BlockSpec((T, DIM), lambda e: (0, 0)),
            pl.BlockSpec((1, INTER, DIM), lambda e: (e, 0, 0)),
            pl.BlockSpec((1, INTER, DIM), lambda e: (e, 0, 0)),
            pl.BlockSpec((1, DIM, INTER), lambda e: (e, 0, 0)),
        ],
        out_specs=pl.BlockSpec((T, DIM), lambda e: (0, 0)),
        out_shape=jax.ShapeDtypeStruct((T, DIM), jnp.float32),
    )(coef, xf, z, w1, w3, w2)

    return y.reshape(shape)
